# baseline (speedup 1.0000x reference)
import functools

import jax
import jax.numpy as jnp
from jax.experimental import pallas as pl
from jax.experimental.pallas import tpu as pltpu

_MERGE_THRESHOLD = 0.85
_MIN_STRENGTH = 0.001
_STRENGTH_DECAY = 0.999
_WRITE_ALPHA = 0.25
_WRITE_BETA = 0.25
_BIG = 1 << 30


def _body(wk_ref, wv_ref, ws_ref, kb_ref, vb_ref, age_ref, st_ref,
          ko_ref, vo_ref, ageo_ref, sto_ref, slot_ref, sim_ref):
    bb, d, k = kb_ref.shape

    wk = wk_ref[...]
    wksq = jnp.sum(wk * wk, axis=1, keepdims=True)
    wk_nrm = jnp.sqrt(wksq) + 1e-6
    wkn3 = (wk / wk_nrm)[:, :, None]

    kb = kb_ref[...]
    dots = jnp.sum(kb * wkn3, axis=1)
    nsq = jnp.sum(kb * kb, axis=1)
    sim = dots / (jnp.sqrt(nsq) + 1e-6)

    best = jnp.max(sim, axis=1, keepdims=True)
    ki = jax.lax.broadcasted_iota(jnp.int32, (bb, k), 1)
    best_idx = jnp.min(jnp.where(sim == best, ki, _BIG), axis=1, keepdims=True)

    age = age_ref[...]
    st = st_ref[...]
    ascore = age + (1.0 - jnp.clip(st, 0.0, 1.0)) * 0.01
    amax = jnp.max(ascore, axis=1, keepdims=True)
    lru = jnp.min(jnp.where(ascore == amax, ki, _BIG), axis=1, keepdims=True)

    slot = jnp.where(best > _MERGE_THRESHOLD, best_idx, lru)
    at_slot = ki == slot

    ws = jnp.clip(ws_ref[...], 0.0, 1.0)
    ageo_ref[...] = jnp.where(at_slot, 0.0, age + 1.0)
    sdec = st * _STRENGTH_DECAY
    prev = jnp.sum(jnp.where(at_slot, sdec, 0.0), axis=1, keepdims=True)
    upd = jnp.clip(prev + ws * (1.0 - prev), _MIN_STRENGTH, 1.0)
    sto_ref[...] = jnp.where(at_slot, upd, sdec)

    sel = at_slot[:, None, :]

    alpha = _WRITE_ALPHA * ws
    oma = 1.0 - alpha
    dots_at = jnp.sum(jnp.where(at_slot, dots, 0.0), axis=1, keepdims=True)
    nsq_at = jnp.sum(jnp.where(at_slot, nsq, 0.0), axis=1, keepdims=True)
    dotw_at = dots_at * wk_nrm
    nk2 = oma * oma * nsq_at + 2.0 * alpha * oma * dotw_at + alpha * alpha * wksq
    rcp_k = 1.0 / (jnp.sqrt(nk2) + 1e-6)

    coef_k = (alpha * wk_nrm)[:, :, None]
    blend_k = (oma[:, :, None] * kb + coef_k * wkn3) * rcp_k[:, :, None]
    ko_ref[...] = jnp.where(sel, blend_k, kb)

    vb = vb_ref[...]
    wv3 = wv_ref[...][:, :, None]
    beta = _WRITE_BETA * ws
    blend_v = (1.0 - beta)[:, :, None] * vb + beta[:, :, None] * wv3
    vo_ref[...] = jnp.where(sel, blend_v, vb)

    slot_ref[...] = slot
    sim_ref[...] = best


@functools.partial(jax.jit, static_argnames=("bb", "interpret"))
def _run(write_key, write_val, write_strength, epi_keys, epi_vals, epi_age,
         epi_strength, bb=64, interpret=False):
    b, k, d = epi_keys.shape
    ekt = epi_keys.transpose(0, 2, 1)
    evt = epi_vals.transpose(0, 2, 1)

    grid = (b // bb,)
    rowd = pl.BlockSpec((bb, d), lambda i: (i, 0))
    rowk = pl.BlockSpec((bb, k), lambda i: (i, 0))
    row1 = pl.BlockSpec((bb, 1), lambda i: (i, 0))
    big = pl.BlockSpec((bb, d, k), lambda i: (i, 0, 0))

    outs = pl.pallas_call(
        _body,
        grid=grid,
        in_specs=[rowd, rowd, row1, big, big, rowk, rowk],
        out_specs=[big, big, rowk, rowk, row1, row1],
        out_shape=[
            jax.ShapeDtypeStruct((b, d, k), jnp.float32),
            jax.ShapeDtypeStruct((b, d, k), jnp.float32),
            jax.ShapeDtypeStruct((b, k), jnp.float32),
            jax.ShapeDtypeStruct((b, k), jnp.float32),
            jax.ShapeDtypeStruct((b, 1), jnp.int32),
            jax.ShapeDtypeStruct((b, 1), jnp.float32),
        ],
        compiler_params=pltpu.CompilerParams(
            dimension_semantics=("arbitrary",)),
        interpret=interpret,
    )(write_key, write_val, write_strength, ekt, evt, epi_age, epi_strength)

    ko, vo, ageo, sto, slot, sim = outs
    return (ko.transpose(0, 2, 1), vo.transpose(0, 2, 1), ageo, sto,
            slot.reshape(b), sim.reshape(b))


def kernel(write_key, write_val, write_strength, epi_keys, epi_vals,
           epi_age, epi_strength):
    return _run(write_key, write_val, write_strength, epi_keys, epi_vals,
                epi_age, epi_strength)
